# baseline (speedup 1.0000x reference)
import functools

import jax
import jax.numpy as jnp
from jax import lax
from jax.experimental import pallas as pl
from jax.experimental.pallas import tpu as pltpu
from jax.experimental.pallas import tpu_sc as plsc

NC = 2
NS = 16
NW = NC * NS
CHUNK = 128


def _tc_pack_uc(UL, UV):
    N, K = UL.shape
    R = 1000
    G = N // R

    def body(a_r, b_r, o_r):
        o_r[:, :K] = a_r[...]
        o_r[:, K:] = b_r[...]

    return pl.pallas_call(
        body,
        grid=(G,),
        in_specs=[pl.BlockSpec((R, K), lambda g: (g, 0)),
                  pl.BlockSpec((R, K), lambda g: (g, 0))],
        out_specs=pl.BlockSpec((R, 2 * K), lambda g: (g, 0)),
        out_shape=jax.ShapeDtypeStruct((N, 2 * K), jnp.float32),
    )(UL, UV)


def _tc_pack_ic(IL, ib):
    N, K = IL.shape
    R = 1000
    G = N // R
    ib3 = ib.reshape(G, 1, R)

    def body(a_r, b_r, o_r):
        o_r[:, :K] = a_r[...]
        o_r[:, K:] = jnp.broadcast_to(b_r[0, 0, :].reshape(R, 1), (R, K))

    return pl.pallas_call(
        body,
        grid=(G,),
        in_specs=[pl.BlockSpec((R, K), lambda g: (g, 0)),
                  pl.BlockSpec((1, 1, R), lambda g: (g, 0, 0))],
        out_specs=pl.BlockSpec((R, 2 * K), lambda g: (g, 0)),
        out_shape=jax.ShapeDtypeStruct((N, 2 * K), jnp.float32),
    )(IL, ib3)


def _sc_gather2(T1, idx1, T2, idx2):
    B = idx1.shape[0]
    F = T1.shape[1]
    bpw = B // NW
    nch = bpw // CHUNK
    mesh = plsc.VectorSubcoreMesh(core_axis_name="c", subcore_axis_name="s")

    @functools.partial(
        pl.kernel,
        out_type=(jax.ShapeDtypeStruct((B, F), jnp.float32),
                  jax.ShapeDtypeStruct((B, F), jnp.float32)),
        mesh=mesh,
        scratch_types=[
            pltpu.VMEM((CHUNK,), jnp.int32),
            pltpu.VMEM((CHUNK,), jnp.int32),
            pltpu.VMEM((CHUNK, F), jnp.float32),
            pltpu.VMEM((CHUNK, F), jnp.float32),
            pltpu.SemaphoreType.DMA,
        ],
        compiler_params=pltpu.CompilerParams(use_tc_tiling_on_sc=False),
    )
    def k(i1_hbm, i2_hbm, t1, t2, o1, o2, i1_c, i2_c, b1, b2, sem):
        cid = lax.axis_index("c")
        sid = lax.axis_index("s")
        wid = sid * NC + cid
        base = wid * bpw
        for c in range(nch):
            sl = pl.ds(base + c * CHUNK, CHUNK)
            pltpu.sync_copy(i1_hbm.at[sl], i1_c)
            pltpu.sync_copy(i2_hbm.at[sl], i2_c)
            cps = [pltpu.async_copy(t1.at[i1_c], b1, sem),
                   pltpu.async_copy(t2.at[i2_c], b2, sem)]
            for cp in cps:
                cp.wait()
            pltpu.sync_copy(b1, o1.at[sl])
            pltpu.sync_copy(b2, o2.at[sl])

    return k(idx1, idx2, T1, T2)


def _sc_gather3(T1, idx1, T2, idx2, idx3):
    B = idx1.shape[0]
    F = T1.shape[1]
    bpw = B // NW
    nch = bpw // CHUNK
    mesh = plsc.VectorSubcoreMesh(core_axis_name="c", subcore_axis_name="s")

    @functools.partial(
        pl.kernel,
        out_type=(jax.ShapeDtypeStruct((B, F), jnp.float32),
                  jax.ShapeDtypeStruct((B, F), jnp.float32),
                  jax.ShapeDtypeStruct((B, F), jnp.float32)),
        mesh=mesh,
        scratch_types=[
            pltpu.VMEM((CHUNK,), jnp.int32),
            pltpu.VMEM((CHUNK,), jnp.int32),
            pltpu.VMEM((CHUNK,), jnp.int32),
            pltpu.VMEM((CHUNK, F), jnp.float32),
            pltpu.VMEM((CHUNK, F), jnp.float32),
            pltpu.VMEM((CHUNK, F), jnp.float32),
            pltpu.SemaphoreType.DMA,
        ],
        compiler_params=pltpu.CompilerParams(use_tc_tiling_on_sc=False),
    )
    def k(i1_hbm, i2_hbm, i3_hbm, t1, t2, o1, o2, o3,
          i1_c, i2_c, i3_c, b1, b2, b3, sem):
        cid = lax.axis_index("c")
        sid = lax.axis_index("s")
        wid = sid * NC + cid
        base = wid * bpw
        for c in range(nch):
            sl = pl.ds(base + c * CHUNK, CHUNK)
            pltpu.sync_copy(i1_hbm.at[sl], i1_c)
            pltpu.sync_copy(i2_hbm.at[sl], i2_c)
            pltpu.sync_copy(i3_hbm.at[sl], i3_c)
            cps = [pltpu.async_copy(t1.at[i1_c], b1, sem),
                   pltpu.async_copy(t2.at[i2_c], b2, sem),
                   pltpu.async_copy(t2.at[i3_c], b3, sem)]
            for cp in cps:
                cp.wait()
            pltpu.sync_copy(b1, o1.at[sl])
            pltpu.sync_copy(b2, o2.at[sl])
            pltpu.sync_copy(b3, o3.at[sl])

    return k(idx1, idx2, idx3, T1, T2)


def _tc_compute(guc, gici, gicj, vfi, vfj, W_proj, beta):
    B, F = guc.shape
    K = W_proj.shape[0]
    BLK = 1024
    NB = B // BLK

    def body(guc_r, gici_r, gicj_r, vfi_r, vfj_r, W_r, beta_r, o_r):
        ul = guc_r[:, :K]
        uv = guc_r[:, K:]
        dil = gici_r[:, :K] - gicj_r[:, :K]
        dib = gici_r[:, K] - gicj_r[:, K]
        dvf = vfi_r[...] - vfj_r[...]
        proj = lax.dot_general(dvf, W_r[...], (((1,), (1,)), ((), ())),
                               preferred_element_type=jnp.float32)
        lat = jnp.sum(ul * dil, axis=1)
        vis = jnp.sum(uv * proj, axis=1)
        bet = jnp.sum(dvf * beta_r[...], axis=1)
        o_r[0, 0, :] = dib + lat + vis + bet

    bf = pl.BlockSpec((BLK, F), lambda b: (b, 0))
    out3 = pl.pallas_call(
        body,
        grid=(NB,),
        in_specs=[bf, bf, bf, bf, bf,
                  pl.BlockSpec((K, F), lambda b: (0, 0)),
                  pl.BlockSpec((1, F), lambda b: (0, 0))],
        out_specs=pl.BlockSpec((1, 1, BLK), lambda b: (b, 0, 0)),
        out_shape=jax.ShapeDtypeStruct((NB, 1, BLK), jnp.float32),
    )(guc, gici, gicj, vfi, vfj, W_proj, beta)
    return out3.reshape(B)


def kernel(trg_batch, U_latent, I_latent, U_visual, W_proj, b_proj,
           beta_dash, user_bias, item_bias, visual_features):
    tb = trg_batch.astype(jnp.int32)
    u_idx = tb[:, 0]
    i_idx = tb[:, 1]
    j_idx = tb[:, 2]
    vfi, vfj = _sc_gather2(visual_features, i_idx, visual_features, j_idx)
    UC = _tc_pack_uc(U_latent, U_visual)
    IC = _tc_pack_ic(I_latent, item_bias)
    guc, gici, gicj = _sc_gather3(UC, u_idx, IC, i_idx, j_idx)
    return _tc_compute(guc, gici, gicj, vfi, vfj, W_proj, beta_dash)


# speedup vs baseline: 1.2717x; 1.2717x over previous
import functools

import jax
import jax.numpy as jnp
from jax import lax
from jax.experimental import pallas as pl
from jax.experimental.pallas import tpu as pltpu
from jax.experimental.pallas import tpu_sc as plsc

NC = 2
NS = 16
NW = NC * NS
CHUNK = 128


def _sc_gather_all(vf, UL, UV, IL, ib2, u_idx, i_idx, j_idx, ibi, ibj):
    B = u_idx.shape[0]
    F = vf.shape[1]
    K = UL.shape[1]
    G = ib2.shape[1]
    bpw = B // NW
    nch = bpw // CHUNK
    mesh = plsc.VectorSubcoreMesh(core_axis_name="c", subcore_axis_name="s")

    @functools.partial(
        pl.kernel,
        out_type=(jax.ShapeDtypeStruct((B, F), jnp.float32),
                  jax.ShapeDtypeStruct((B, F), jnp.float32),
                  jax.ShapeDtypeStruct((B, K), jnp.float32),
                  jax.ShapeDtypeStruct((B, K), jnp.float32),
                  jax.ShapeDtypeStruct((B, K), jnp.float32),
                  jax.ShapeDtypeStruct((B, K), jnp.float32),
                  jax.ShapeDtypeStruct((B, G), jnp.float32),
                  jax.ShapeDtypeStruct((B, G), jnp.float32)),
        mesh=mesh,
        scratch_types=[
            pltpu.VMEM((CHUNK,), jnp.int32),
            pltpu.VMEM((CHUNK,), jnp.int32),
            pltpu.VMEM((CHUNK,), jnp.int32),
            pltpu.VMEM((CHUNK,), jnp.int32),
            pltpu.VMEM((CHUNK,), jnp.int32),
            pltpu.VMEM((CHUNK, F), jnp.float32),
            pltpu.VMEM((CHUNK, F), jnp.float32),
            pltpu.VMEM((CHUNK, K), jnp.float32),
            pltpu.VMEM((CHUNK, K), jnp.float32),
            pltpu.VMEM((CHUNK, K), jnp.float32),
            pltpu.VMEM((CHUNK, K), jnp.float32),
            pltpu.VMEM((CHUNK, G), jnp.float32),
            pltpu.VMEM((CHUNK, G), jnp.float32),
            pltpu.SemaphoreType.DMA,
        ],
        compiler_params=pltpu.CompilerParams(use_tc_tiling_on_sc=False),
    )
    def k(u_hbm, i_hbm, j_hbm, ibi_hbm, ibj_hbm,
          vf_t, ul_t, uv_t, il_t, ib_t,
          o_vfi, o_vfj, o_ul, o_uv, o_ili, o_ilj, o_ibi, o_ibj,
          u_c, i_c, j_c, ibi_c, ibj_c,
          b_vfi, b_vfj, b_ul, b_uv, b_ili, b_ilj, b_ibi, b_ibj, sem):
        cid = lax.axis_index("c")
        sid = lax.axis_index("s")
        wid = sid * NC + cid
        base = wid * bpw
        for c in range(nch):
            sl = pl.ds(base + c * CHUNK, CHUNK)
            pltpu.sync_copy(u_hbm.at[sl], u_c)
            pltpu.sync_copy(i_hbm.at[sl], i_c)
            pltpu.sync_copy(j_hbm.at[sl], j_c)
            pltpu.sync_copy(ibi_hbm.at[sl], ibi_c)
            pltpu.sync_copy(ibj_hbm.at[sl], ibj_c)
            cps = [pltpu.async_copy(vf_t.at[i_c], b_vfi, sem),
                   pltpu.async_copy(vf_t.at[j_c], b_vfj, sem),
                   pltpu.async_copy(ul_t.at[u_c], b_ul, sem),
                   pltpu.async_copy(uv_t.at[u_c], b_uv, sem),
                   pltpu.async_copy(il_t.at[i_c], b_ili, sem),
                   pltpu.async_copy(il_t.at[j_c], b_ilj, sem),
                   pltpu.async_copy(ib_t.at[ibi_c], b_ibi, sem),
                   pltpu.async_copy(ib_t.at[ibj_c], b_ibj, sem)]
            for cp in cps:
                cp.wait()
            pltpu.sync_copy(b_vfi, o_vfi.at[sl])
            pltpu.sync_copy(b_vfj, o_vfj.at[sl])
            pltpu.sync_copy(b_ul, o_ul.at[sl])
            pltpu.sync_copy(b_uv, o_uv.at[sl])
            pltpu.sync_copy(b_ili, o_ili.at[sl])
            pltpu.sync_copy(b_ilj, o_ilj.at[sl])
            pltpu.sync_copy(b_ibi, o_ibi.at[sl])
            pltpu.sync_copy(b_ibj, o_ibj.at[sl])

    return k(u_idx, i_idx, j_idx, ibi, ibj, vf, UL, UV, IL, ib2)


def _tc_compute(vfi, vfj, gul, guv, gili, gilj, gibi, gibj, ohi, ohj,
                W_proj, beta):
    B, F = vfi.shape
    K = gul.shape[1]
    BLK = 1024
    NB = B // BLK

    def body(vfi_r, vfj_r, gul_r, guv_r, gili_r, gilj_r, gibi_r, gibj_r,
             ohi_r, ohj_r, W_r, beta_r, o_r):
        dvf = vfi_r[...] - vfj_r[...]
        proj = lax.dot_general(dvf, W_r[...], (((1,), (1,)), ((), ())),
                               preferred_element_type=jnp.float32)
        lat = jnp.sum(gul_r[...] * (gili_r[...] - gilj_r[...]), axis=1)
        vis = jnp.sum(guv_r[...] * proj, axis=1)
        bet = jnp.sum(dvf * beta_r[...], axis=1)
        dib = jnp.sum(gibi_r[...] * ohi_r[...] - gibj_r[...] * ohj_r[...],
                      axis=1)
        o_r[0, 0, :] = dib + lat + vis + bet

    G = gibi.shape[1]
    bF = pl.BlockSpec((BLK, F), lambda b: (b, 0))
    bK = pl.BlockSpec((BLK, K), lambda b: (b, 0))
    bG = pl.BlockSpec((BLK, G), lambda b: (b, 0))
    out3 = pl.pallas_call(
        body,
        grid=(NB,),
        in_specs=[bF, bF, bK, bK, bK, bK, bG, bG, bG, bG,
                  pl.BlockSpec((K, F), lambda b: (0, 0)),
                  pl.BlockSpec((1, F), lambda b: (0, 0))],
        out_specs=pl.BlockSpec((1, 1, BLK), lambda b: (b, 0, 0)),
        out_shape=jax.ShapeDtypeStruct((NB, 1, BLK), jnp.float32),
    )(vfi, vfj, gul, guv, gili, gilj, gibi, gibj, ohi, ohj, W_proj, beta)
    return out3.reshape(B)


def kernel(trg_batch, U_latent, I_latent, U_visual, W_proj, b_proj,
           beta_dash, user_bias, item_bias, visual_features):
    tb = trg_batch.astype(jnp.int32)
    u_idx = tb[:, 0]
    i_idx = tb[:, 1]
    j_idx = tb[:, 2]
    ibi = i_idx >> 4
    ibj = j_idx >> 4
    lanes = jnp.arange(16, dtype=jnp.int32)[None, :]
    ohi = ((i_idx & 15)[:, None] == lanes).astype(jnp.float32)
    ohj = ((j_idx & 15)[:, None] == lanes).astype(jnp.float32)
    ib2 = item_bias.reshape(-1, 16)
    vfi, vfj, gul, guv, gili, gilj, gibi, gibj = _sc_gather_all(
        visual_features, U_latent, U_visual, I_latent, ib2,
        u_idx, i_idx, j_idx, ibi, ibj)
    return _tc_compute(vfi, vfj, gul, guv, gili, gilj, gibi, gibj,
                       ohi, ohj, W_proj, beta_dash)


# speedup vs baseline: 1.3257x; 1.0425x over previous
import functools

import jax
import jax.numpy as jnp
from jax import lax
from jax.experimental import pallas as pl
from jax.experimental.pallas import tpu as pltpu
from jax.experimental.pallas import tpu_sc as plsc

NC = 2
NS = 16
NW = NC * NS
CHUNK = 128


def _sc_gather2(T1, idx1, T2, idx2):
    B = idx1.shape[0]
    F = T1.shape[1]
    bpw = B // NW
    nch = bpw // CHUNK
    mesh = plsc.VectorSubcoreMesh(core_axis_name="c", subcore_axis_name="s")

    @functools.partial(
        pl.kernel,
        out_type=(jax.ShapeDtypeStruct((B, F), jnp.float32),
                  jax.ShapeDtypeStruct((B, F), jnp.float32)),
        mesh=mesh,
        scratch_types=[
            pltpu.VMEM((CHUNK,), jnp.int32),
            pltpu.VMEM((CHUNK,), jnp.int32),
            pltpu.VMEM((CHUNK, F), jnp.float32),
            pltpu.VMEM((CHUNK, F), jnp.float32),
            pltpu.SemaphoreType.DMA,
        ],
        compiler_params=pltpu.CompilerParams(use_tc_tiling_on_sc=True),
    )
    def k(i1_hbm, i2_hbm, t1, t2, o1, o2, i1_c, i2_c, b1, b2, sem):
        cid = lax.axis_index("c")
        sid = lax.axis_index("s")
        wid = sid * NC + cid
        base = wid * bpw
        for c in range(nch):
            sl = pl.ds(base + c * CHUNK, CHUNK)
            pltpu.sync_copy(i1_hbm.at[sl], i1_c)
            pltpu.sync_copy(i2_hbm.at[sl], i2_c)
            cps = [pltpu.async_copy(t1.at[i1_c], b1, sem),
                   pltpu.async_copy(t2.at[i2_c], b2, sem)]
            for cp in cps:
                cp.wait()
            pltpu.sync_copy(b1, o1.at[sl])
            pltpu.sync_copy(b2, o2.at[sl])

    return k(idx1, idx2, T1, T2)


def _sc_gather3(T1, idx1, T2, idx2, idx3):
    B = idx1.shape[0]
    F = T1.shape[1]
    bpw = B // NW
    nch = bpw // CHUNK
    mesh = plsc.VectorSubcoreMesh(core_axis_name="c", subcore_axis_name="s")

    @functools.partial(
        pl.kernel,
        out_type=(jax.ShapeDtypeStruct((B, F), jnp.float32),
                  jax.ShapeDtypeStruct((B, F), jnp.float32),
                  jax.ShapeDtypeStruct((B, F), jnp.float32)),
        mesh=mesh,
        scratch_types=[
            pltpu.VMEM((CHUNK,), jnp.int32),
            pltpu.VMEM((CHUNK,), jnp.int32),
            pltpu.VMEM((CHUNK,), jnp.int32),
            pltpu.VMEM((CHUNK, F), jnp.float32),
            pltpu.VMEM((CHUNK, F), jnp.float32),
            pltpu.VMEM((CHUNK, F), jnp.float32),
            pltpu.SemaphoreType.DMA,
        ],
        compiler_params=pltpu.CompilerParams(use_tc_tiling_on_sc=True),
    )
    def k(i1_hbm, i2_hbm, i3_hbm, t1, t2, o1, o2, o3,
          i1_c, i2_c, i3_c, b1, b2, b3, sem):
        cid = lax.axis_index("c")
        sid = lax.axis_index("s")
        wid = sid * NC + cid
        base = wid * bpw
        for c in range(nch):
            sl = pl.ds(base + c * CHUNK, CHUNK)
            pltpu.sync_copy(i1_hbm.at[sl], i1_c)
            pltpu.sync_copy(i2_hbm.at[sl], i2_c)
            pltpu.sync_copy(i3_hbm.at[sl], i3_c)
            cps = [pltpu.async_copy(t1.at[i1_c], b1, sem),
                   pltpu.async_copy(t2.at[i2_c], b2, sem),
                   pltpu.async_copy(t2.at[i3_c], b3, sem)]
            for cp in cps:
                cp.wait()
            pltpu.sync_copy(b1, o1.at[sl])
            pltpu.sync_copy(b2, o2.at[sl])
            pltpu.sync_copy(b3, o3.at[sl])

    return k(idx1, idx2, idx3, T1, T2)


def _tc_compute(guc, gici, gicj, vfi, vfj, W_proj, beta):
    B, F = guc.shape
    K = W_proj.shape[0]
    BLK = 1024
    NB = B // BLK

    def body(guc_r, gici_r, gicj_r, vfi_r, vfj_r, W_r, beta_r, o_r):
        ul = guc_r[:, :K]
        uv = guc_r[:, K:]
        dil = gici_r[:, :K] - gicj_r[:, :K]
        dib = gici_r[:, K] - gicj_r[:, K]
        dvf = vfi_r[...] - vfj_r[...]
        proj = lax.dot_general(dvf, W_r[...], (((1,), (1,)), ((), ())),
                               preferred_element_type=jnp.float32)
        lat = jnp.sum(ul * dil, axis=1)
        vis = jnp.sum(uv * proj, axis=1)
        bet = jnp.sum(dvf * beta_r[...], axis=1)
        o_r[0, 0, :] = dib + lat + vis + bet

    bf = pl.BlockSpec((BLK, F), lambda b: (b, 0))
    out3 = pl.pallas_call(
        body,
        grid=(NB,),
        in_specs=[bf, bf, bf, bf, bf,
                  pl.BlockSpec((K, F), lambda b: (0, 0)),
                  pl.BlockSpec((1, F), lambda b: (0, 0))],
        out_specs=pl.BlockSpec((1, 1, BLK), lambda b: (b, 0, 0)),
        out_shape=jax.ShapeDtypeStruct((NB, 1, BLK), jnp.float32),
    )(guc, gici, gicj, vfi, vfj, W_proj, beta)
    return out3.reshape(B)


def kernel(trg_batch, U_latent, I_latent, U_visual, W_proj, b_proj,
           beta_dash, user_bias, item_bias, visual_features):
    tb = trg_batch.astype(jnp.int32)
    u_idx = tb[:, 0]
    i_idx = tb[:, 1]
    j_idx = tb[:, 2]
    vfi, vfj = _sc_gather2(visual_features, i_idx, visual_features, j_idx)
    N, K = U_latent.shape
    UC = jnp.concatenate([U_latent, U_visual], axis=1)
    IC = jnp.concatenate(
        [I_latent, jnp.broadcast_to(item_bias[:, None], (N, K))], axis=1)
    guc, gici, gicj = _sc_gather3(UC, u_idx, IC, i_idx, j_idx)
    return _tc_compute(guc, gici, gicj, vfi, vfj, W_proj, beta_dash)


# speedup vs baseline: 1.4130x; 1.0658x over previous
import functools

import jax
import jax.numpy as jnp
from jax import lax
from jax.experimental import pallas as pl
from jax.experimental.pallas import tpu as pltpu
from jax.experimental.pallas import tpu_sc as plsc

NC = 2
NS = 16
NW = NC * NS
CHUNK = 128


def _tc_pack_uc(ULt, UVt):
    K, N = ULt.shape
    C = 1024
    G = -(-N // C)

    def body(a_r, b_r, o_r):
        o_r[:, :K] = jnp.transpose(a_r[...])
        o_r[:, K:] = jnp.transpose(b_r[...])

    return pl.pallas_call(
        body,
        grid=(G,),
        in_specs=[pl.BlockSpec((K, C), lambda g: (0, g)),
                  pl.BlockSpec((K, C), lambda g: (0, g))],
        out_specs=pl.BlockSpec((C, 2 * K), lambda g: (g, 0)),
        out_shape=jax.ShapeDtypeStruct((N, 2 * K), jnp.float32),
    )(ULt, UVt)


def _tc_pack_ic(ILt, ib):
    K, N = ILt.shape
    C = 1024
    G = -(-N // C)
    ib2 = ib.reshape(1, N)

    def body(a_r, b_r, o_r):
        o_r[:, :K] = jnp.transpose(a_r[...])
        o_r[:, K:] = jnp.broadcast_to(b_r[0, :].reshape(C, 1), (C, K))

    return pl.pallas_call(
        body,
        grid=(G,),
        in_specs=[pl.BlockSpec((K, C), lambda g: (0, g)),
                  pl.BlockSpec((1, C), lambda g: (0, g))],
        out_specs=pl.BlockSpec((C, 2 * K), lambda g: (g, 0)),
        out_shape=jax.ShapeDtypeStruct((N, 2 * K), jnp.float32),
    )(ILt, ib2)


def _sc_gather2(T1, idx1, T2, idx2):
    B = idx1.shape[0]
    F = T1.shape[1]
    bpw = B // NW
    nch = bpw // CHUNK
    mesh = plsc.VectorSubcoreMesh(core_axis_name="c", subcore_axis_name="s")

    @functools.partial(
        pl.kernel,
        out_type=(jax.ShapeDtypeStruct((B, F), jnp.float32),
                  jax.ShapeDtypeStruct((B, F), jnp.float32)),
        mesh=mesh,
        scratch_types=[
            pltpu.VMEM((CHUNK,), jnp.int32),
            pltpu.VMEM((CHUNK,), jnp.int32),
            pltpu.VMEM((CHUNK, F), jnp.float32),
            pltpu.VMEM((CHUNK, F), jnp.float32),
            pltpu.SemaphoreType.DMA,
        ],
        compiler_params=pltpu.CompilerParams(use_tc_tiling_on_sc=True),
    )
    def k(i1_hbm, i2_hbm, t1, t2, o1, o2, i1_c, i2_c, b1, b2, sem):
        cid = lax.axis_index("c")
        sid = lax.axis_index("s")
        wid = sid * NC + cid
        base = wid * bpw
        for c in range(nch):
            sl = pl.ds(base + c * CHUNK, CHUNK)
            pltpu.sync_copy(i1_hbm.at[sl], i1_c)
            pltpu.sync_copy(i2_hbm.at[sl], i2_c)
            cps = [pltpu.async_copy(t1.at[i1_c], b1, sem),
                   pltpu.async_copy(t2.at[i2_c], b2, sem)]
            for cp in cps:
                cp.wait()
            pltpu.sync_copy(b1, o1.at[sl])
            pltpu.sync_copy(b2, o2.at[sl])

    return k(idx1, idx2, T1, T2)


def _sc_gather3(T1, idx1, T2, idx2, idx3):
    B = idx1.shape[0]
    F = T1.shape[1]
    bpw = B // NW
    nch = bpw // CHUNK
    mesh = plsc.VectorSubcoreMesh(core_axis_name="c", subcore_axis_name="s")

    @functools.partial(
        pl.kernel,
        out_type=(jax.ShapeDtypeStruct((B, F), jnp.float32),
                  jax.ShapeDtypeStruct((B, F), jnp.float32),
                  jax.ShapeDtypeStruct((B, F), jnp.float32)),
        mesh=mesh,
        scratch_types=[
            pltpu.VMEM((CHUNK,), jnp.int32),
            pltpu.VMEM((CHUNK,), jnp.int32),
            pltpu.VMEM((CHUNK,), jnp.int32),
            pltpu.VMEM((CHUNK, F), jnp.float32),
            pltpu.VMEM((CHUNK, F), jnp.float32),
            pltpu.VMEM((CHUNK, F), jnp.float32),
            pltpu.SemaphoreType.DMA,
        ],
        compiler_params=pltpu.CompilerParams(use_tc_tiling_on_sc=True),
    )
    def k(i1_hbm, i2_hbm, i3_hbm, t1, t2, o1, o2, o3,
          i1_c, i2_c, i3_c, b1, b2, b3, sem):
        cid = lax.axis_index("c")
        sid = lax.axis_index("s")
        wid = sid * NC + cid
        base = wid * bpw
        for c in range(nch):
            sl = pl.ds(base + c * CHUNK, CHUNK)
            pltpu.sync_copy(i1_hbm.at[sl], i1_c)
            pltpu.sync_copy(i2_hbm.at[sl], i2_c)
            pltpu.sync_copy(i3_hbm.at[sl], i3_c)
            cps = [pltpu.async_copy(t1.at[i1_c], b1, sem),
                   pltpu.async_copy(t2.at[i2_c], b2, sem),
                   pltpu.async_copy(t2.at[i3_c], b3, sem)]
            for cp in cps:
                cp.wait()
            pltpu.sync_copy(b1, o1.at[sl])
            pltpu.sync_copy(b2, o2.at[sl])
            pltpu.sync_copy(b3, o3.at[sl])

    return k(idx1, idx2, idx3, T1, T2)


def _tc_compute(guc, gici, gicj, vfi, vfj, W_proj, beta):
    B, F = guc.shape
    K = W_proj.shape[0]
    BLK = 1024
    NB = B // BLK

    def body(guc_r, gici_r, gicj_r, vfi_r, vfj_r, W_r, beta_r, o_r):
        ul = guc_r[:, :K]
        uv = guc_r[:, K:]
        dil = gici_r[:, :K] - gicj_r[:, :K]
        dib = gici_r[:, K] - gicj_r[:, K]
        dvf = vfi_r[...] - vfj_r[...]
        proj = lax.dot_general(dvf, W_r[...], (((1,), (1,)), ((), ())),
                               preferred_element_type=jnp.float32)
        lat = jnp.sum(ul * dil, axis=1)
        vis = jnp.sum(uv * proj, axis=1)
        bet = jnp.sum(dvf * beta_r[...], axis=1)
        o_r[0, 0, :] = dib + lat + vis + bet

    bf = pl.BlockSpec((BLK, F), lambda b: (b, 0))
    out3 = pl.pallas_call(
        body,
        grid=(NB,),
        in_specs=[bf, bf, bf, bf, bf,
                  pl.BlockSpec((K, F), lambda b: (0, 0)),
                  pl.BlockSpec((1, F), lambda b: (0, 0))],
        out_specs=pl.BlockSpec((1, 1, BLK), lambda b: (b, 0, 0)),
        out_shape=jax.ShapeDtypeStruct((NB, 1, BLK), jnp.float32),
    )(guc, gici, gicj, vfi, vfj, W_proj, beta)
    return out3.reshape(B)


def kernel(trg_batch, U_latent, I_latent, U_visual, W_proj, b_proj,
           beta_dash, user_bias, item_bias, visual_features):
    tb = trg_batch.astype(jnp.int32)
    u_idx = tb[:, 0]
    i_idx = tb[:, 1]
    j_idx = tb[:, 2]
    vfi, vfj = _sc_gather2(visual_features, i_idx, visual_features, j_idx)
    UC = _tc_pack_uc(U_latent.T, U_visual.T)
    IC = _tc_pack_ic(I_latent.T, item_bias)
    guc, gici, gicj = _sc_gather3(UC, u_idx, IC, i_idx, j_idx)
    return _tc_compute(guc, gici, gicj, vfi, vfj, W_proj, beta_dash)


# speedup vs baseline: 2.1101x; 1.4933x over previous
import functools

import jax
import jax.numpy as jnp
from jax import lax
from jax.experimental import pallas as pl
from jax.experimental.pallas import tpu as pltpu
from jax.experimental.pallas import tpu_sc as plsc

NC = 2
NS = 16
NW = NC * NS
CHUNK = 128


def _mxu_t(a, eye):
    return lax.dot_general(a, eye, (((0,), (0,)), ((), ())),
                           preferred_element_type=jnp.float32)


def _tc_pack(ULt, UVt, ILt, ib):
    K, N = ULt.shape
    C = 4096
    G = -(-N // C)
    ib2 = ib.reshape(1, N)
    eye = jnp.eye(K, dtype=jnp.float32)

    def body(a_r, b_r, c_r, d_r, e_r, uc_r, ic_r):
        uc_r[:, :K] = _mxu_t(a_r[...], e_r[...])
        uc_r[:, K:] = _mxu_t(b_r[...], e_r[...])
        ic_r[:, :K] = _mxu_t(c_r[...], e_r[...])
        ic_r[:, K:] = jnp.broadcast_to(d_r[0, :].reshape(C, 1), (C, K))

    bt = pl.BlockSpec((K, C), lambda g: (0, g))
    bo = pl.BlockSpec((C, 2 * K), lambda g: (g, 0))
    return pl.pallas_call(
        body,
        grid=(G,),
        in_specs=[bt, bt, bt,
                  pl.BlockSpec((1, C), lambda g: (0, g)),
                  pl.BlockSpec((K, K), lambda g: (0, 0))],
        out_specs=(bo, bo),
        out_shape=(jax.ShapeDtypeStruct((N, 2 * K), jnp.float32),
                   jax.ShapeDtypeStruct((N, 2 * K), jnp.float32)),
    )(ULt, UVt, ILt, ib2, eye)


def _sc_gather2(T1, idx1, T2, idx2):
    B = idx1.shape[0]
    F = T1.shape[1]
    bpw = B // NW
    nch = bpw // CHUNK
    mesh = plsc.VectorSubcoreMesh(core_axis_name="c", subcore_axis_name="s")

    @functools.partial(
        pl.kernel,
        out_type=(jax.ShapeDtypeStruct((B, F), jnp.float32),
                  jax.ShapeDtypeStruct((B, F), jnp.float32)),
        mesh=mesh,
        scratch_types=[
            pltpu.VMEM((CHUNK,), jnp.int32),
            pltpu.VMEM((CHUNK,), jnp.int32),
            pltpu.VMEM((CHUNK, F), jnp.float32),
            pltpu.VMEM((CHUNK, F), jnp.float32),
            pltpu.SemaphoreType.DMA,
        ],
        compiler_params=pltpu.CompilerParams(use_tc_tiling_on_sc=True),
    )
    def k(i1_hbm, i2_hbm, t1, t2, o1, o2, i1_c, i2_c, b1, b2, sem):
        cid = lax.axis_index("c")
        sid = lax.axis_index("s")
        wid = sid * NC + cid
        base = wid * bpw
        for c in range(nch):
            sl = pl.ds(base + c * CHUNK, CHUNK)
            pltpu.sync_copy(i1_hbm.at[sl], i1_c)
            pltpu.sync_copy(i2_hbm.at[sl], i2_c)
            cps = [pltpu.async_copy(t1.at[i1_c], b1, sem),
                   pltpu.async_copy(t2.at[i2_c], b2, sem)]
            for cp in cps:
                cp.wait()
            pltpu.sync_copy(b1, o1.at[sl])
            pltpu.sync_copy(b2, o2.at[sl])

    return k(idx1, idx2, T1, T2)


def _sc_gather3(T1, idx1, T2, idx2, idx3):
    B = idx1.shape[0]
    F = T1.shape[1]
    bpw = B // NW
    nch = bpw // CHUNK
    mesh = plsc.VectorSubcoreMesh(core_axis_name="c", subcore_axis_name="s")

    @functools.partial(
        pl.kernel,
        out_type=(jax.ShapeDtypeStruct((B, F), jnp.float32),
                  jax.ShapeDtypeStruct((B, F), jnp.float32),
                  jax.ShapeDtypeStruct((B, F), jnp.float32)),
        mesh=mesh,
        scratch_types=[
            pltpu.VMEM((CHUNK,), jnp.int32),
            pltpu.VMEM((CHUNK,), jnp.int32),
            pltpu.VMEM((CHUNK,), jnp.int32),
            pltpu.VMEM((CHUNK, F), jnp.float32),
            pltpu.VMEM((CHUNK, F), jnp.float32),
            pltpu.VMEM((CHUNK, F), jnp.float32),
            pltpu.SemaphoreType.DMA,
        ],
        compiler_params=pltpu.CompilerParams(use_tc_tiling_on_sc=True),
    )
    def k(i1_hbm, i2_hbm, i3_hbm, t1, t2, o1, o2, o3,
          i1_c, i2_c, i3_c, b1, b2, b3, sem):
        cid = lax.axis_index("c")
        sid = lax.axis_index("s")
        wid = sid * NC + cid
        base = wid * bpw
        for c in range(nch):
            sl = pl.ds(base + c * CHUNK, CHUNK)
            pltpu.sync_copy(i1_hbm.at[sl], i1_c)
            pltpu.sync_copy(i2_hbm.at[sl], i2_c)
            pltpu.sync_copy(i3_hbm.at[sl], i3_c)
            cps = [pltpu.async_copy(t1.at[i1_c], b1, sem),
                   pltpu.async_copy(t2.at[i2_c], b2, sem),
                   pltpu.async_copy(t2.at[i3_c], b3, sem)]
            for cp in cps:
                cp.wait()
            pltpu.sync_copy(b1, o1.at[sl])
            pltpu.sync_copy(b2, o2.at[sl])
            pltpu.sync_copy(b3, o3.at[sl])

    return k(idx1, idx2, idx3, T1, T2)


def _tc_compute(guc, gici, gicj, vfi, vfj, W_proj, beta):
    B, F = guc.shape
    K = W_proj.shape[0]
    BLK = 1024
    NB = B // BLK

    def body(guc_r, gici_r, gicj_r, vfi_r, vfj_r, W_r, beta_r, o_r):
        ul = guc_r[:, :K]
        uv = guc_r[:, K:]
        dil = gici_r[:, :K] - gicj_r[:, :K]
        dib = gici_r[:, K] - gicj_r[:, K]
        dvf = vfi_r[...] - vfj_r[...]
        proj = lax.dot_general(dvf, W_r[...], (((1,), (1,)), ((), ())),
                               preferred_element_type=jnp.float32)
        lat = jnp.sum(ul * dil, axis=1)
        vis = jnp.sum(uv * proj, axis=1)
        bet = jnp.sum(dvf * beta_r[...], axis=1)
        o_r[0, 0, :] = dib + lat + vis + bet

    bf = pl.BlockSpec((BLK, F), lambda b: (b, 0))
    out3 = pl.pallas_call(
        body,
        grid=(NB,),
        in_specs=[bf, bf, bf, bf, bf,
                  pl.BlockSpec((K, F), lambda b: (0, 0)),
                  pl.BlockSpec((1, F), lambda b: (0, 0))],
        out_specs=pl.BlockSpec((1, 1, BLK), lambda b: (b, 0, 0)),
        out_shape=jax.ShapeDtypeStruct((NB, 1, BLK), jnp.float32),
    )(guc, gici, gicj, vfi, vfj, W_proj, beta)
    return out3.reshape(B)


def kernel(trg_batch, U_latent, I_latent, U_visual, W_proj, b_proj,
           beta_dash, user_bias, item_bias, visual_features):
    tb = trg_batch.astype(jnp.int32)
    u_idx = tb[:, 0]
    i_idx = tb[:, 1]
    j_idx = tb[:, 2]
    vfi, vfj = _sc_gather2(visual_features, i_idx, visual_features, j_idx)
    UC, IC = _tc_pack(U_latent.T, U_visual.T, I_latent.T, item_bias)
    guc, gici, gicj = _sc_gather3(UC, u_idx, IC, i_idx, j_idx)
    return _tc_compute(guc, gici, gicj, vfi, vfj, W_proj, beta_dash)


# speedup vs baseline: 2.1590x; 1.0232x over previous
import functools

import jax
import jax.numpy as jnp
from jax import lax
from jax.experimental import pallas as pl
from jax.experimental.pallas import tpu as pltpu
from jax.experimental.pallas import tpu_sc as plsc

NC = 2
NS = 16
NW = NC * NS
CHUNK = 128


def _mxu_t(a, eye):
    return lax.dot_general(a, eye, (((0,), (0,)), ((), ())),
                           preferred_element_type=jnp.float32)


def _bf16_bits(x):
    b = lax.bitcast_convert_type(x, jnp.uint32)
    return (b + jnp.uint32(0x7FFF) + ((b >> 16) & jnp.uint32(1))) >> 16


def _unpack_lo(w):
    return lax.bitcast_convert_type(w << 16, jnp.float32)


def _unpack_hi(w):
    return lax.bitcast_convert_type(w & jnp.uint32(0xFFFF0000), jnp.float32)


def _tc_pack(ULt, UVt, ILt, ib):
    K, N = ULt.shape
    C = 4096
    G = -(-N // C)
    ib2 = ib.reshape(1, N)
    eye = jnp.eye(K, dtype=jnp.float32)

    def body(a_r, b_r, c_r, d_r, e_r, t_r):
        ul = _bf16_bits(_mxu_t(a_r[...], e_r[...]))
        uv = _bf16_bits(_mxu_t(b_r[...], e_r[...]))
        il = _bf16_bits(_mxu_t(c_r[...], e_r[...]))
        ibv = _bf16_bits(jnp.broadcast_to(d_r[0, :].reshape(C, 1), (C, K)))
        t_r[:, :K] = ul | (uv << 16)
        t_r[:, K:] = il | (ibv << 16)

    bt = pl.BlockSpec((K, C), lambda g: (0, g))
    return pl.pallas_call(
        body,
        grid=(G,),
        in_specs=[bt, bt, bt,
                  pl.BlockSpec((1, C), lambda g: (0, g)),
                  pl.BlockSpec((K, K), lambda g: (0, 0))],
        out_specs=pl.BlockSpec((C, 2 * K), lambda g: (g, 0)),
        out_shape=jax.ShapeDtypeStruct((N, 2 * K), jnp.uint32),
    )(ULt, UVt, ILt, ib2, eye)


def _sc_gather2(T1, idx1, T2, idx2):
    B = idx1.shape[0]
    F = T1.shape[1]
    dt = T1.dtype
    bpw = B // NW
    nch = bpw // CHUNK
    mesh = plsc.VectorSubcoreMesh(core_axis_name="c", subcore_axis_name="s")

    @functools.partial(
        pl.kernel,
        out_type=(jax.ShapeDtypeStruct((B, F), dt),
                  jax.ShapeDtypeStruct((B, F), dt)),
        mesh=mesh,
        scratch_types=[
            pltpu.VMEM((CHUNK,), jnp.int32),
            pltpu.VMEM((CHUNK,), jnp.int32),
            pltpu.VMEM((CHUNK, F), dt),
            pltpu.VMEM((CHUNK, F), dt),
            pltpu.SemaphoreType.DMA,
        ],
        compiler_params=pltpu.CompilerParams(use_tc_tiling_on_sc=True),
    )
    def k(i1_hbm, i2_hbm, t1, t2, o1, o2, i1_c, i2_c, b1, b2, sem):
        cid = lax.axis_index("c")
        sid = lax.axis_index("s")
        wid = sid * NC + cid
        base = wid * bpw
        for c in range(nch):
            sl = pl.ds(base + c * CHUNK, CHUNK)
            pltpu.sync_copy(i1_hbm.at[sl], i1_c)
            pltpu.sync_copy(i2_hbm.at[sl], i2_c)
            cps = [pltpu.async_copy(t1.at[i1_c], b1, sem),
                   pltpu.async_copy(t2.at[i2_c], b2, sem)]
            for cp in cps:
                cp.wait()
            pltpu.sync_copy(b1, o1.at[sl])
            pltpu.sync_copy(b2, o2.at[sl])

    return k(idx1, idx2, T1, T2)


def _sc_gather3(T, idx1, idx2, idx3):
    B = idx1.shape[0]
    F = T.shape[1]
    dt = T.dtype
    bpw = B // NW
    nch = bpw // CHUNK
    mesh = plsc.VectorSubcoreMesh(core_axis_name="c", subcore_axis_name="s")

    @functools.partial(
        pl.kernel,
        out_type=(jax.ShapeDtypeStruct((B, F), dt),
                  jax.ShapeDtypeStruct((B, F), dt),
                  jax.ShapeDtypeStruct((B, F), dt)),
        mesh=mesh,
        scratch_types=[
            pltpu.VMEM((CHUNK,), jnp.int32),
            pltpu.VMEM((CHUNK,), jnp.int32),
            pltpu.VMEM((CHUNK,), jnp.int32),
            pltpu.VMEM((CHUNK, F), dt),
            pltpu.VMEM((CHUNK, F), dt),
            pltpu.VMEM((CHUNK, F), dt),
            pltpu.SemaphoreType.DMA,
        ],
        compiler_params=pltpu.CompilerParams(use_tc_tiling_on_sc=True),
    )
    def k(i1_hbm, i2_hbm, i3_hbm, t1, o1, o2, o3,
          i1_c, i2_c, i3_c, b1, b2, b3, sem):
        cid = lax.axis_index("c")
        sid = lax.axis_index("s")
        wid = sid * NC + cid
        base = wid * bpw
        for c in range(nch):
            sl = pl.ds(base + c * CHUNK, CHUNK)
            pltpu.sync_copy(i1_hbm.at[sl], i1_c)
            pltpu.sync_copy(i2_hbm.at[sl], i2_c)
            pltpu.sync_copy(i3_hbm.at[sl], i3_c)
            cps = [pltpu.async_copy(t1.at[i1_c], b1, sem),
                   pltpu.async_copy(t1.at[i2_c], b2, sem),
                   pltpu.async_copy(t1.at[i3_c], b3, sem)]
            for cp in cps:
                cp.wait()
            pltpu.sync_copy(b1, o1.at[sl])
            pltpu.sync_copy(b2, o2.at[sl])
            pltpu.sync_copy(b3, o3.at[sl])

    return k(idx1, idx2, idx3, T)


def _tc_compute(gu, gi, gj, vfi, vfj, W_proj, beta):
    B, F = gu.shape
    K = W_proj.shape[0]
    BLK = 1024
    NB = B // BLK

    def body(gu_r, gi_r, gj_r, vfi_r, vfj_r, W_r, beta_r, o_r):
        wu = gu_r[:, :K]
        ul = _unpack_lo(wu)
        uv = _unpack_hi(wu)
        wi = gi_r[:, K:]
        wj = gj_r[:, K:]
        dil = _unpack_lo(wi) - _unpack_lo(wj)
        dib = _unpack_hi(wi[:, 0]) - _unpack_hi(wj[:, 0])
        dvf = vfi_r[...] - vfj_r[...]
        proj = lax.dot_general(dvf, W_r[...], (((1,), (1,)), ((), ())),
                               preferred_element_type=jnp.float32)
        lat = jnp.sum(ul * dil, axis=1)
        vis = jnp.sum(uv * proj, axis=1)
        bet = jnp.sum(dvf * beta_r[...], axis=1)
        o_r[0, 0, :] = dib + lat + vis + bet

    bf = pl.BlockSpec((BLK, F), lambda b: (b, 0))
    out3 = pl.pallas_call(
        body,
        grid=(NB,),
        in_specs=[bf, bf, bf, bf, bf,
                  pl.BlockSpec((K, F), lambda b: (0, 0)),
                  pl.BlockSpec((1, F), lambda b: (0, 0))],
        out_specs=pl.BlockSpec((1, 1, BLK), lambda b: (b, 0, 0)),
        out_shape=jax.ShapeDtypeStruct((NB, 1, BLK), jnp.float32),
    )(gu, gi, gj, vfi, vfj, W_proj, beta)
    return out3.reshape(B)


def kernel(trg_batch, U_latent, I_latent, U_visual, W_proj, b_proj,
           beta_dash, user_bias, item_bias, visual_features):
    tb = trg_batch.astype(jnp.int32)
    u_idx = tb[:, 0]
    i_idx = tb[:, 1]
    j_idx = tb[:, 2]
    vfi, vfj = _sc_gather2(visual_features, i_idx, visual_features, j_idx)
    T = _tc_pack(U_latent.T, U_visual.T, I_latent.T, item_bias)
    gu, gi, gj = _sc_gather3(T, u_idx, i_idx, j_idx)
    return _tc_compute(gu, gi, gj, vfi, vfj, W_proj, beta_dash)


# speedup vs baseline: 2.1748x; 1.0073x over previous
import functools

import jax
import jax.numpy as jnp
from jax import lax
from jax.experimental import pallas as pl
from jax.experimental.pallas import tpu as pltpu
from jax.experimental.pallas import tpu_sc as plsc

NC = 2
NS = 16
NW = NC * NS
CHUNK = 128


def _mxu_t(a, eye):
    return lax.dot_general(a, eye, (((0,), (0,)), ((), ())),
                           preferred_element_type=jnp.float32)


def _bf16_bits(x):
    b = lax.bitcast_convert_type(x, jnp.uint32)
    return (b + jnp.uint32(0x7FFF) + ((b >> 16) & jnp.uint32(1))) >> 16


def _unpack_lo(w):
    return lax.bitcast_convert_type(w << 16, jnp.float32)


def _unpack_hi(w):
    return lax.bitcast_convert_type(w & jnp.uint32(0xFFFF0000), jnp.float32)


def _tc_pack(ULt, UVt, ILt, ib):
    K, N = ULt.shape
    C = 4096
    G = -(-N // C)
    ib2 = ib.reshape(1, N)
    eye = jnp.eye(K, dtype=jnp.float32)

    def body(a_r, b_r, c_r, d_r, e_r, t_r):
        ul = _bf16_bits(_mxu_t(a_r[...], e_r[...]))
        uv = _bf16_bits(_mxu_t(b_r[...], e_r[...]))
        il = _bf16_bits(_mxu_t(c_r[...], e_r[...]))
        ibv = _bf16_bits(jnp.broadcast_to(d_r[0, :].reshape(C, 1), (C, K)))
        t_r[:, :K] = ul | (uv << 16)
        t_r[:, K:] = il | (ibv << 16)

    bt = pl.BlockSpec((K, C), lambda g: (0, g))
    return pl.pallas_call(
        body,
        grid=(G,),
        in_specs=[bt, bt, bt,
                  pl.BlockSpec((1, C), lambda g: (0, g)),
                  pl.BlockSpec((K, K), lambda g: (0, 0))],
        out_specs=pl.BlockSpec((C, 2 * K), lambda g: (g, 0)),
        out_shape=jax.ShapeDtypeStruct((N, 2 * K), jnp.uint32),
        compiler_params=pltpu.CompilerParams(
            dimension_semantics=("parallel",)),
    )(ULt, UVt, ILt, ib2, eye)


def _sc_gather2(T1, idx1, T2, idx2):
    B = idx1.shape[0]
    F = T1.shape[1]
    dt = T1.dtype
    bpw = B // NW
    nch = bpw // CHUNK
    mesh = plsc.VectorSubcoreMesh(core_axis_name="c", subcore_axis_name="s")

    @functools.partial(
        pl.kernel,
        out_type=(jax.ShapeDtypeStruct((B, F), dt),
                  jax.ShapeDtypeStruct((B, F), dt)),
        mesh=mesh,
        scratch_types=[
            pltpu.VMEM((CHUNK,), jnp.int32),
            pltpu.VMEM((CHUNK,), jnp.int32),
            pltpu.VMEM((CHUNK, F), dt),
            pltpu.VMEM((CHUNK, F), dt),
            pltpu.SemaphoreType.DMA,
        ],
        compiler_params=pltpu.CompilerParams(use_tc_tiling_on_sc=True),
    )
    def k(i1_hbm, i2_hbm, t1, t2, o1, o2, i1_c, i2_c, b1, b2, sem):
        cid = lax.axis_index("c")
        sid = lax.axis_index("s")
        wid = sid * NC + cid
        base = wid * bpw
        for c in range(nch):
            sl = pl.ds(base + c * CHUNK, CHUNK)
            pltpu.sync_copy(i1_hbm.at[sl], i1_c)
            pltpu.sync_copy(i2_hbm.at[sl], i2_c)
            cps = [pltpu.async_copy(t1.at[i1_c], b1, sem),
                   pltpu.async_copy(t2.at[i2_c], b2, sem)]
            for cp in cps:
                cp.wait()
            pltpu.sync_copy(b1, o1.at[sl])
            pltpu.sync_copy(b2, o2.at[sl])

    return k(idx1, idx2, T1, T2)


def _sc_gather3(T, idx1, idx2, idx3):
    B = idx1.shape[0]
    F = T.shape[1]
    dt = T.dtype
    bpw = B // NW
    nch = bpw // CHUNK
    mesh = plsc.VectorSubcoreMesh(core_axis_name="c", subcore_axis_name="s")

    @functools.partial(
        pl.kernel,
        out_type=(jax.ShapeDtypeStruct((B, F), dt),
                  jax.ShapeDtypeStruct((B, F), dt),
                  jax.ShapeDtypeStruct((B, F), dt)),
        mesh=mesh,
        scratch_types=[
            pltpu.VMEM((CHUNK,), jnp.int32),
            pltpu.VMEM((CHUNK,), jnp.int32),
            pltpu.VMEM((CHUNK,), jnp.int32),
            pltpu.VMEM((CHUNK, F), dt),
            pltpu.VMEM((CHUNK, F), dt),
            pltpu.VMEM((CHUNK, F), dt),
            pltpu.SemaphoreType.DMA,
        ],
        compiler_params=pltpu.CompilerParams(use_tc_tiling_on_sc=True),
    )
    def k(i1_hbm, i2_hbm, i3_hbm, t1, o1, o2, o3,
          i1_c, i2_c, i3_c, b1, b2, b3, sem):
        cid = lax.axis_index("c")
        sid = lax.axis_index("s")
        wid = sid * NC + cid
        base = wid * bpw
        for c in range(nch):
            sl = pl.ds(base + c * CHUNK, CHUNK)
            pltpu.sync_copy(i1_hbm.at[sl], i1_c)
            pltpu.sync_copy(i2_hbm.at[sl], i2_c)
            pltpu.sync_copy(i3_hbm.at[sl], i3_c)
            cps = [pltpu.async_copy(t1.at[i1_c], b1, sem),
                   pltpu.async_copy(t1.at[i2_c], b2, sem),
                   pltpu.async_copy(t1.at[i3_c], b3, sem)]
            for cp in cps:
                cp.wait()
            pltpu.sync_copy(b1, o1.at[sl])
            pltpu.sync_copy(b2, o2.at[sl])
            pltpu.sync_copy(b3, o3.at[sl])

    return k(idx1, idx2, idx3, T)


def _tc_compute(gu, gi, gj, vfi, vfj, W_proj, beta):
    B, F = gu.shape
    K = W_proj.shape[0]
    BLK = 1024
    NB = B // BLK

    def body(gu_r, gi_r, gj_r, vfi_r, vfj_r, W_r, beta_r, o_r):
        wu = gu_r[:, :K]
        ul = _unpack_lo(wu)
        uv = _unpack_hi(wu)
        wi = gi_r[:, K:]
        wj = gj_r[:, K:]
        dil = _unpack_lo(wi) - _unpack_lo(wj)
        dib = _unpack_hi(wi[:, 0]) - _unpack_hi(wj[:, 0])
        dvf = vfi_r[...] - vfj_r[...]
        proj = lax.dot_general(dvf, W_r[...], (((1,), (1,)), ((), ())),
                               preferred_element_type=jnp.float32)
        lat = jnp.sum(ul * dil, axis=1)
        vis = jnp.sum(uv * proj, axis=1)
        bet = jnp.sum(dvf * beta_r[...], axis=1)
        o_r[0, 0, :] = dib + lat + vis + bet

    bf = pl.BlockSpec((BLK, F), lambda b: (b, 0))
    out3 = pl.pallas_call(
        body,
        grid=(NB,),
        in_specs=[bf, bf, bf, bf, bf,
                  pl.BlockSpec((K, F), lambda b: (0, 0)),
                  pl.BlockSpec((1, F), lambda b: (0, 0))],
        out_specs=pl.BlockSpec((1, 1, BLK), lambda b: (b, 0, 0)),
        out_shape=jax.ShapeDtypeStruct((NB, 1, BLK), jnp.float32),
        compiler_params=pltpu.CompilerParams(
            dimension_semantics=("parallel",)),
    )(gu, gi, gj, vfi, vfj, W_proj, beta)
    return out3.reshape(B)


def kernel(trg_batch, U_latent, I_latent, U_visual, W_proj, b_proj,
           beta_dash, user_bias, item_bias, visual_features):
    tb = trg_batch.astype(jnp.int32)
    u_idx = tb[:, 0]
    i_idx = tb[:, 1]
    j_idx = tb[:, 2]
    vfi, vfj = _sc_gather2(visual_features, i_idx, visual_features, j_idx)
    T = _tc_pack(U_latent.T, U_visual.T, I_latent.T, item_bias)
    gu, gi, gj = _sc_gather3(T, u_idx, i_idx, j_idx)
    return _tc_compute(gu, gi, gj, vfi, vfj, W_proj, beta_dash)
